# baseline (speedup 1.0000x reference)
import jax
import jax.numpy as jnp
from jax import lax
from jax.experimental import pallas as pl
from jax.experimental.pallas import tpu as pltpu
from jax.experimental.pallas import tpu_sc as plsc

VOCAB = 1000000
EMB_DIM = 64
BATCH = 4096
HIST = 50

NC = 2
NS = 16
NW = NC * NS

TOTAL = BATCH * HIST
CHUNK = 128
NCHUNKS = TOTAL // CHUNK
CPW = NCHUNKS // NW

_mesh = plsc.VectorSubcoreMesh(core_axis_name="c", subcore_axis_name="s")


def _body(idx_hbm, table_hbm, out_hbm, idx_v, rows0, rows1, gsem0, gsem1):
    wid = lax.axis_index("s") * NC + lax.axis_index("c")
    c0 = wid * CPW

    pltpu.sync_copy(idx_hbm.at[wid], idx_v)

    rows = (rows0, rows1)
    gsems = (gsem0, gsem1)

    def start_gather(j, b):
        pltpu.async_copy(table_hbm.at[idx_v.at[j]], rows[b], gsems[b])

    def wait_gather(j, b):
        pltpu.make_async_copy(table_hbm.at[idx_v.at[j]], rows[b], gsems[b]).wait()

    start_gather(0, 0)

    @pl.loop(0, CPW, step=2)
    def step(j0):
        wait_gather(j0, 0)
        start_gather(j0 + 1, 1)
        pltpu.sync_copy(rows[0], out_hbm.at[c0 + j0])
        wait_gather(j0 + 1, 1)

        @pl.when(j0 + 2 < CPW)
        def _():
            start_gather(j0 + 2, 0)

        pltpu.sync_copy(rows[1], out_hbm.at[c0 + j0 + 1])


_gather = pl.kernel(
    _body,
    out_type=jax.ShapeDtypeStruct((NCHUNKS, CHUNK, EMB_DIM), jnp.float32),
    mesh=_mesh,
    scratch_types=[
        pltpu.VMEM((CPW, CHUNK), jnp.int32),
        pltpu.VMEM((CHUNK, EMB_DIM), jnp.float32),
        pltpu.VMEM((CHUNK, EMB_DIM), jnp.float32),
        pltpu.SemaphoreType.DMA,
        pltpu.SemaphoreType.DMA,
    ],
    compiler_params=pltpu.CompilerParams(use_tc_tiling_on_sc=False),
)


def kernel(src_seq, emb_weight):
    idx = src_seq.astype(jnp.int32).reshape(NW, CPW, CHUNK)
    out = _gather(idx, emb_weight)
    return out.reshape(BATCH, HIST, EMB_DIM)


# speedup vs baseline: 1.0303x; 1.0303x over previous
import jax
import jax.numpy as jnp
from jax import lax
from jax.experimental import pallas as pl
from jax.experimental.pallas import tpu as pltpu
from jax.experimental.pallas import tpu_sc as plsc

VOCAB = 1000000
EMB_DIM = 64
BATCH = 4096
HIST = 50

NC = 2
NS = 16
NW = NC * NS

TOTAL = BATCH * HIST
CHUNK = 128
NCHUNKS = TOTAL // CHUNK
CPW = NCHUNKS // NW

NBUF = 5
AHEAD = NBUF - 1

_mesh = plsc.VectorSubcoreMesh(core_axis_name="c", subcore_axis_name="s")


def _body(idx_hbm, table_hbm, out_hbm, idx_v, rows, gsems, osems):
    wid = lax.axis_index("s") * NC + lax.axis_index("c")
    c0 = wid * CPW

    pltpu.sync_copy(idx_hbm.at[wid], idx_v)

    def start_gather(j, b):
        pltpu.async_copy(table_hbm.at[idx_v.at[j]], rows[b], gsems[b])

    def wait_gather(j, b):
        pltpu.make_async_copy(table_hbm.at[idx_v.at[j]], rows[b], gsems[b]).wait()

    def start_out(j, b):
        pltpu.async_copy(rows[b], out_hbm.at[c0 + j], osems[b])

    def wait_out(j, b):
        pltpu.make_async_copy(rows[b], out_hbm.at[c0 + j], osems[b]).wait()

    for b in range(AHEAD):
        start_gather(b, b)

    @pl.loop(0, CPW, step=NBUF)
    def step(j0):
        for b in range(NBUF):
            j = j0 + b
            jn = j + AHEAD
            bn = (b + AHEAD) % NBUF

            @pl.when(jn < CPW)
            def _():
                if b == 0:
                    @pl.when(j >= 1)
                    def _():
                        wait_out(j - 1, bn)
                else:
                    wait_out(j - 1, bn)
                start_gather(jn, bn)

            wait_gather(j, b)
            start_out(j, b)

    for b in range(NBUF):
        wait_out(CPW - NBUF + b, b)


_gather = pl.kernel(
    _body,
    out_type=jax.ShapeDtypeStruct((NCHUNKS, CHUNK, EMB_DIM), jnp.float32),
    mesh=_mesh,
    scratch_types=[
        pltpu.VMEM((CPW, CHUNK), jnp.int32),
        [pltpu.VMEM((CHUNK, EMB_DIM), jnp.float32) for _ in range(NBUF)],
        [pltpu.SemaphoreType.DMA for _ in range(NBUF)],
        [pltpu.SemaphoreType.DMA for _ in range(NBUF)],
    ],
    compiler_params=pltpu.CompilerParams(use_tc_tiling_on_sc=False),
)


def kernel(src_seq, emb_weight):
    idx = src_seq.astype(jnp.int32).reshape(NW, CPW, CHUNK)
    out = _gather(idx, emb_weight)
    return out.reshape(BATCH, HIST, EMB_DIM)
